# baseline (speedup 1.0000x reference)
import functools

import jax
import jax.numpy as jnp
from jax import lax
from jax.experimental import pallas as pl
from jax.experimental.pallas import tpu as pltpu
from jax.experimental.pallas import tpu_sc as plsc

MAX_REL = 1024
L = 1024
D = 64
NC = 2
NS = 16
NW = NC * NS
NGROUPS = 4
SLABS_PER_G = 8
COLS = 15 * 128


PADC = 16 * 128


LAYERS_PER_STEP = 32


def _build_body(t_ref, o_ref):
    r4 = pl.program_id(0)
    t = t_ref[...]
    for k in range(LAYERS_PER_STEP):
        r = r4 * LAYERS_PER_STEP + k
        c = jnp.where(r == 0, 128, r)
        o_ref[k] = pltpu.roll(t, PADC - c, 2)[:, :, :COLS]


_build_tc = pl.pallas_call(
    _build_body,
    grid=(128 // LAYERS_PER_STEP,),
    in_specs=[pl.BlockSpec((8, 8, PADC), lambda r: (0, 0, 0))],
    out_specs=pl.BlockSpec((LAYERS_PER_STEP, 8, 8, COLS), lambda r: (r, 0, 0, 0)),
    out_shape=jax.ShapeDtypeStruct((128, 8, 8, COLS), jnp.float32),
    compiler_params=pltpu.CompilerParams(
        dimension_semantics=("arbitrary",),
    ),
)


@functools.partial(
    pl.kernel,
    mesh=plsc.VectorSubcoreMesh(core_axis_name="c", subcore_axis_name="s"),
    out_type=jax.ShapeDtypeStruct((L, 8, 8, 8, 128), jnp.float32),
    scratch_types=[
        pltpu.VMEM((8, 8, COLS), jnp.float32),
        pltpu.SemaphoreType.DMA,
    ],
)
def _rpe_sc(stack_hbm, out_hbm, layer_buf, sem):
    cid = lax.axis_index("c")
    sid = lax.axis_index("s")
    wid = cid * NS + sid

    for g in range(NGROUPS):
        layer = lax.rem(2 * 128 - wid - 32 * g, 128)
        pltpu.sync_copy(stack_hbm.at[layer], layer_buf)
        copies = []
        for tt in range(SLABS_PER_G):
            i = wid + 32 * g + 128 * tt
            mm0 = 7 - tt
            for jt in range(8):
                copies.append(
                    pltpu.async_copy(
                        layer_buf.at[:, :, pl.ds(128 * (mm0 + jt), 128)],
                        out_hbm.at[i, :, jt],
                        sem,
                    )
                )
        for c in copies:
            c.wait()


def kernel(length, relative_attention_bias):
    del length
    t3 = relative_attention_bias.T.reshape(8, 8, 2 * MAX_REL + 1)
    t3p = t3[:, :, :PADC]
    k2 = _rpe_sc(_build_tc(t3p))
    return k2.transpose(0, 2, 4, 1, 3).reshape(L, L, D)


# speedup vs baseline: 1.0069x; 1.0069x over previous
import functools

import jax
import jax.numpy as jnp
from jax import lax
from jax.experimental import pallas as pl
from jax.experimental.pallas import tpu as pltpu
from jax.experimental.pallas import tpu_sc as plsc

MAX_REL = 1024
L = 1024
D = 64
NC = 2
NS = 16
NW = NC * NS
NGROUPS = 4
SLABS_PER_G = 8
COLS = 15 * 128


PADC = 16 * 128


LAYERS_PER_STEP = 16


def _build_body(t_ref, o_ref):
    r4 = pl.program_id(0)
    t = t_ref[...]
    for k in range(LAYERS_PER_STEP):
        r = r4 * LAYERS_PER_STEP + k
        c = jnp.where(r == 0, 128, r)
        o_ref[k] = pltpu.roll(t, PADC - c, 2)[:, :, :COLS]


_build_tc = pl.pallas_call(
    _build_body,
    grid=(128 // LAYERS_PER_STEP,),
    in_specs=[pl.BlockSpec((8, 8, PADC), lambda r: (0, 0, 0))],
    out_specs=pl.BlockSpec((LAYERS_PER_STEP, 8, 8, COLS), lambda r: (r, 0, 0, 0)),
    out_shape=jax.ShapeDtypeStruct((128, 8, 8, COLS), jnp.float32),
    compiler_params=pltpu.CompilerParams(
        dimension_semantics=("arbitrary",),
    ),
)


@functools.partial(
    pl.kernel,
    mesh=plsc.VectorSubcoreMesh(core_axis_name="c", subcore_axis_name="s"),
    out_type=jax.ShapeDtypeStruct((L, 8, 8, 8, 128), jnp.float32),
    scratch_types=[
        pltpu.VMEM((8, 8, COLS), jnp.float32),
        pltpu.SemaphoreType.DMA,
    ],
)
def _rpe_sc(stack_hbm, out_hbm, layer_buf, sem):
    cid = lax.axis_index("c")
    sid = lax.axis_index("s")
    wid = cid * NS + sid

    for g in range(NGROUPS):
        layer = lax.rem(2 * 128 - wid - 32 * g, 128)
        pltpu.sync_copy(stack_hbm.at[layer], layer_buf)
        copies = []
        for tt in range(SLABS_PER_G):
            i = wid + 32 * g + 128 * tt
            mm0 = 7 - tt
            for jt in range(8):
                copies.append(
                    pltpu.async_copy(
                        layer_buf.at[:, :, pl.ds(128 * (mm0 + jt), 128)],
                        out_hbm.at[i, :, jt],
                        sem,
                    )
                )
        for c in copies:
            c.wait()


def kernel(length, relative_attention_bias):
    del length
    t3 = relative_attention_bias.T.reshape(8, 8, 2 * MAX_REL + 1)
    t3p = t3[:, :, :PADC]
    k2 = _rpe_sc(_build_tc(t3p))
    return k2.transpose(0, 2, 4, 1, 3).reshape(L, L, D)
